# baseline (speedup 1.0000x reference)
import jax
import jax.numpy as jnp
from jax.experimental import pallas as pl
from jax.experimental.pallas import tpu as pltpu


def _dense_body(hs_ref, hm_ref, cs_ref, has_ref, x_ref, c_ref,
                wi_ref, ui_ref, bi_ref, ufw_ref, ufb_ref,
                hnew_ref, cnew_ref):
    hs = hs_ref[...]
    hm = hm_ref[...]
    cs = cs_ref[...]
    has_msg = has_ref[...] > 0.0
    hcomb = jnp.concatenate([hs, hm], axis=1)
    dn = (((1,), (1,)), ((), ()))
    f = jax.nn.sigmoid(
        jax.lax.dot_general(hcomb, ufw_ref[...], dn,
                            preferred_element_type=jnp.float32)
        + ufb_ref[...])
    iou_red = jax.lax.dot_general(hcomb, ui_ref[...], dn,
                                  preferred_element_type=jnp.float32)
    iou_leaf = jax.lax.dot_general(x_ref[...], wi_ref[...], dn,
                                   preferred_element_type=jnp.float32)
    iou = jnp.where(has_msg[:, :1], iou_red, iou_leaf) + bi_ref[...]
    i = jax.nn.sigmoid(iou[:, 0:128])
    o = jax.nn.sigmoid(iou[:, 128:256])
    u = jnp.tanh(iou[:, 256:384])
    c_data = jnp.where(has_msg, f * cs, c_ref[...])
    c_new = i * u + c_data
    hnew_ref[...] = o * jnp.tanh(c_new)
    cnew_ref[...] = c_new


def _dense_apply(hs, hm, cs, has, x, c, W_iou, U_iou, b_iou, U_f_w, U_f_b):
    n = x.shape[0]
    B = 2000
    grid = (n // B,)
    row_spec = pl.BlockSpec((B, 128), lambda i: (i, 0))
    full = lambda a: pl.BlockSpec(a.shape, lambda i: tuple(0 for _ in a.shape))
    return pl.pallas_call(
        _dense_body,
        grid=grid,
        in_specs=[row_spec, row_spec, row_spec, row_spec, row_spec, row_spec,
                  full(W_iou), full(U_iou), full(b_iou), full(U_f_w),
                  pl.BlockSpec((1, 128), lambda i: (0, 0))],
        out_specs=[row_spec, row_spec],
        out_shape=[jax.ShapeDtypeStruct((n, 128), jnp.float32),
                   jax.ShapeDtypeStruct((n, 128), jnp.float32)],
    )(hs, hm, cs, has, x, c, W_iou, U_iou, b_iou, U_f_w,
      U_f_b.reshape(1, 128))


def kernel(x, h, c, edge_index, W_iou, U_iou, b_iou, U_f_w, U_f_b):
    src = edge_index[0]
    dst = edge_index[1]
    n = x.shape[0]
    h_src = h[src]
    c_src = c[src]
    deg = jax.ops.segment_sum(jnp.ones((src.shape[0],), dtype=x.dtype), dst,
                              num_segments=n)
    h_sum = jax.ops.segment_sum(h_src, dst, num_segments=n)
    h_max = jax.ops.segment_max(h_src, dst, num_segments=n)
    h_max = jnp.where((deg > 0)[:, None], h_max, 0.0)
    c_sum = jax.ops.segment_sum(c_src, dst, num_segments=n)
    has = jnp.broadcast_to(deg[:, None], (n, 128))
    return _dense_apply(h_sum, h_max, c_sum, has, x, c,
                        W_iou, U_iou, b_iou, U_f_w, U_f_b)


# speedup vs baseline: 1.3155x; 1.3155x over previous
import jax
import jax.numpy as jnp
from jax import lax
from jax.experimental import pallas as pl
from jax.experimental.pallas import tpu as pltpu
from jax.experimental.pallas import tpu_sc as plsc

N = 10000
E = 320000
H = 128
NW = 32
NPW = 320
NPAD = NW * NPW
CH = 800
NCH = E // CH


def _segment_reduce_sc(src, dst, h, c):
    mesh = plsc.VectorSubcoreMesh(core_axis_name="c", subcore_axis_name="s")

    @pl.kernel(
        out_type=[
            jax.ShapeDtypeStruct((NPAD * H,), jnp.float32),
            jax.ShapeDtypeStruct((NPAD * H,), jnp.float32),
            jax.ShapeDtypeStruct((NPAD * H,), jnp.float32),
            jax.ShapeDtypeStruct((NPAD,), jnp.int32),
        ],
        mesh=mesh,
        compiler_params=pltpu.CompilerParams(needs_layout_passes=False),
        scratch_types=[
            pltpu.VMEM((NPW * H,), jnp.float32),
            pltpu.VMEM((NPW * H,), jnp.float32),
            pltpu.VMEM((NPW * H,), jnp.float32),
            pltpu.VMEM((NPW,), jnp.int32),
            pltpu.VMEM((CH,), jnp.int32),
            pltpu.VMEM((CH,), jnp.int32),
            pltpu.VMEM((CH + 16,), jnp.int32),
            pltpu.VMEM((CH + 16,), jnp.int32),
            pltpu.VMEM((16, H), jnp.float32),
            pltpu.VMEM((16, H), jnp.float32),
        ],
    )
    def sc_kernel(src_hbm, dst_hbm, h_hbm, c_hbm,
                  hs_hbm, hm_hbm, cs_hbm, fl_hbm,
                  acc_s, acc_m, acc_c, flag_v, dstb, srcb,
                  msrc, mloc, rowh, rowc):
        wid = lax.axis_index("s") * 2 + lax.axis_index("c")
        lo = wid * NPW

        zero16 = jnp.zeros((16,), jnp.float32)
        ninf16 = jnp.full((16,), -jnp.inf, jnp.float32)
        zero16i = jnp.zeros((16,), jnp.int32)
        one16i = jnp.ones((16,), jnp.int32)

        @pl.loop(0, NPW * H, step=16)
        def _(i):
            acc_s[pl.ds(i, 16)] = zero16
            acc_m[pl.ds(i, 16)] = ninf16
            acc_c[pl.ds(i, 16)] = zero16

        @pl.loop(0, NPW, step=16)
        def _(i):
            flag_v[pl.ds(i, 16)] = zero16i

        @pl.loop(0, CH + 16, step=16)
        def _(i):
            msrc[pl.ds(i, 16)] = zero16i

        @pl.loop(0, NCH)
        def _(ci):
            base = ci * CH
            pltpu.sync_copy(dst_hbm.at[pl.ds(base, CH)], dstb)
            pltpu.sync_copy(src_hbm.at[pl.ds(base, CH)], srcb)

            def fstep(i, cnt):
                d = dstb[pl.ds(i * 16, 16)]
                s = srcb[pl.ds(i * 16, 16)]
                loc = d - lo
                m = (loc >= 0) & (loc < NPW)
                plsc.store_compressed(msrc.at[pl.ds(cnt, 16)], s, mask=m)
                plsc.store_compressed(mloc.at[pl.ds(cnt, 16)], loc, mask=m)
                plsc.store_scatter(flag_v, [loc], one16i, mask=m)
                return cnt + jnp.sum(jnp.where(m, 1, 0))

            cnt = lax.fori_loop(0, CH // 16, fstep, 0)

            lane = lax.iota(jnp.int32, 16)

            def gbody(g, _):
                gi = g * 16
                pltpu.sync_copy(h_hbm.at[msrc.at[pl.ds(gi, 16)]], rowh)
                pltpu.sync_copy(c_hbm.at[msrc.at[pl.ds(gi, 16)]], rowc)
                lvec = mloc[pl.ds(gi, 16)]

                def ebody(j, _):
                    dloc = jnp.sum(jnp.where(lane == j, lvec, 0))
                    b = dloc * H
                    for k in range(8):
                        sl = pl.ds(b + k * 16, 16)
                        rh = rowh[j, pl.ds(k * 16, 16)]
                        plsc.addupdate(acc_s.at[sl], rh)
                        acc_m[sl] = jnp.maximum(acc_m[sl], rh)
                        rc = rowc[j, pl.ds(k * 16, 16)]
                        plsc.addupdate(acc_c.at[sl], rc)
                    return 0

                lax.fori_loop(0, jnp.minimum(cnt - gi, 16), ebody, 0)
                return 0

            lax.fori_loop(0, (cnt + 15) // 16, gbody, 0)

        pltpu.sync_copy(acc_s, hs_hbm.at[pl.ds(lo * H, NPW * H)])
        pltpu.sync_copy(acc_m, hm_hbm.at[pl.ds(lo * H, NPW * H)])
        pltpu.sync_copy(acc_c, cs_hbm.at[pl.ds(lo * H, NPW * H)])
        pltpu.sync_copy(flag_v, fl_hbm.at[pl.ds(lo, NPW)])

    return sc_kernel(src, dst, h, c)


def _dense_body(hs_ref, hm_ref, cs_ref, fl_ref, x_ref, c_ref,
                wi_ref, ui_ref, bi_ref, ufw_ref, ufb_ref,
                hnew_ref, cnew_ref):
    has_msg = fl_ref[...] > 0
    hm = jnp.where(has_msg, hm_ref[...], 0.0)
    hcomb = jnp.concatenate([hs_ref[...], hm], axis=1)
    dn = (((1,), (1,)), ((), ()))
    f = jax.nn.sigmoid(
        jax.lax.dot_general(hcomb, ufw_ref[...], dn,
                            preferred_element_type=jnp.float32)
        + ufb_ref[...])
    iou_red = jax.lax.dot_general(hcomb, ui_ref[...], dn,
                                  preferred_element_type=jnp.float32)
    iou_leaf = jax.lax.dot_general(x_ref[...], wi_ref[...], dn,
                                   preferred_element_type=jnp.float32)
    iou = jnp.where(has_msg, iou_red, iou_leaf) + bi_ref[...]
    i = jax.nn.sigmoid(iou[:, 0:128])
    o = jax.nn.sigmoid(iou[:, 128:256])
    u = jnp.tanh(iou[:, 256:384])
    c_data = jnp.where(has_msg, f * cs_ref[...], c_ref[...])
    c_new = i * u + c_data
    hnew_ref[...] = o * jnp.tanh(c_new)
    cnew_ref[...] = c_new


def _dense_apply(hs, hm, cs, fl, x, c, W_iou, U_iou, b_iou, U_f_w, U_f_b):
    B = 400
    grid = (N // B,)
    row_spec = pl.BlockSpec((B, H), lambda i: (i, 0))
    flag_spec = pl.BlockSpec((B, 1), lambda i: (i, 0))
    full = lambda a: pl.BlockSpec(a.shape, lambda i: tuple(0 for _ in a.shape))
    return pl.pallas_call(
        _dense_body,
        grid=grid,
        in_specs=[row_spec, row_spec, row_spec, flag_spec, row_spec, row_spec,
                  full(W_iou), full(U_iou), full(b_iou), full(U_f_w),
                  pl.BlockSpec((1, H), lambda i: (0, 0))],
        out_specs=[row_spec, row_spec],
        out_shape=[jax.ShapeDtypeStruct((N, H), jnp.float32),
                   jax.ShapeDtypeStruct((N, H), jnp.float32)],
    )(hs, hm, cs, fl, x, c, W_iou, U_iou, b_iou, U_f_w,
      U_f_b.reshape(1, H))


def kernel(x, h, c, edge_index, W_iou, U_iou, b_iou, U_f_w, U_f_b):
    src = edge_index[0]
    dst = edge_index[1]
    hs, hm, cs, fl = _segment_reduce_sc(src, dst, h, c)
    hs = hs.reshape(NPAD, H)[:N]
    hm = hm.reshape(NPAD, H)[:N]
    cs = cs.reshape(NPAD, H)[:N]
    has = fl[:N].reshape(N, 1)
    return _dense_apply(hs, hm, cs, has, x, c,
                        W_iou, U_iou, b_iou, U_f_w, U_f_b)


# speedup vs baseline: 2.2971x; 1.7462x over previous
import jax
import jax.numpy as jnp
from jax import lax
from jax.experimental import pallas as pl
from jax.experimental.pallas import tpu as pltpu
from jax.experimental.pallas import tpu_sc as plsc

N = 10000
E = 320000
H = 128
NW = 32
NPW = 320
HNPW = 160
NPAD = NW * NPW
CH = 1280
NCH = E // CH
G = 64
MB = 1504


def _segment_reduce_sc(ei, h, c):
    mesh = plsc.VectorSubcoreMesh(core_axis_name="c", subcore_axis_name="s")

    @pl.kernel(
        out_type=[
            jax.ShapeDtypeStruct((NPAD, H), jnp.float32),
            jax.ShapeDtypeStruct((NPAD, H), jnp.float32),
            jax.ShapeDtypeStruct((NPAD, H), jnp.float32),
        ],
        mesh=mesh,
        compiler_params=pltpu.CompilerParams(needs_layout_passes=False),
        scratch_types=[
            pltpu.VMEM((HNPW, H), jnp.float32),
            pltpu.VMEM((HNPW, H), jnp.float32),
            pltpu.VMEM((HNPW, H), jnp.float32),
            pltpu.VMEM((2, CH), jnp.int32),
            pltpu.VMEM((2, CH), jnp.int32),
            pltpu.VMEM((MB,), jnp.int32),
            pltpu.VMEM((MB,), jnp.int32),
            pltpu.VMEM((G, H), jnp.float32),
            pltpu.VMEM((G, H), jnp.float32),
            pltpu.SemaphoreType.DMA,
            pltpu.SemaphoreType.DMA,
            pltpu.SemaphoreType.DMA,
            pltpu.SemaphoreType.DMA,
        ],
    )
    def sc_kernel(ei_hbm, h_hbm, c_hbm, hs_hbm, hm_hbm, cs_hbm,
                  acc_s, acc_m, acc_c, chA, chB, msrc, mloc,
                  rowh, rowc, sg1, sg2, sA, sB):
        s = lax.axis_index("s")
        wid = s * 2 + lax.axis_index("c")
        lo0 = wid * NPW

        zero16 = jnp.zeros((16,), jnp.float32)
        ninf16 = jnp.full((16,), -jnp.inf, jnp.float32)
        zero16i = jnp.zeros((16,), jnp.int32)
        lane = lax.iota(jnp.int32, 16)

        @pl.loop(0, MB, step=16)
        def _(i):
            msrc[pl.ds(i, 16)] = zero16i
            mloc[pl.ds(i, 16)] = zero16i

        def process(off, count):
            off = pl.multiple_of(off, 16)

            def kgroup(k, _):
                lvec = mloc[pl.ds(off + k * 16, 16)]

                def ebody(j, _):
                    dloc = jnp.sum(jnp.where(lane == j, lvec, 0))
                    r = k * 16 + j
                    for k2 in range(8):
                        sl = pl.ds(k2 * 16, 16)
                        rh = rowh[r, sl]
                        plsc.addupdate(acc_s.at[dloc, sl], rh)
                        acc_m[dloc, sl] = jnp.maximum(acc_m[dloc, sl], rh)
                        plsc.addupdate(acc_c.at[dloc, sl], rowc[r, sl])
                    return 0

                lax.fori_loop(0, jnp.clip(count - k * 16, 0, 16), ebody, 0)
                return 0

            lax.fori_loop(0, G // 16, kgroup, 0)

        def fire(off):
            off = pl.multiple_of(off, 16)
            pltpu.async_copy(h_hbm.at[msrc.at[pl.ds(off, G)]], rowh, sg1)
            pltpu.async_copy(c_hbm.at[msrc.at[pl.ds(off, G)]], rowc, sg2)

        def wait_gathers():
            pltpu.make_async_copy(
                h_hbm.at[msrc.at[pl.ds(0, G)]], rowh, sg1).wait()
            pltpu.make_async_copy(
                c_hbm.at[msrc.at[pl.ds(0, G)]], rowc, sg2).wait()

        def filter_chunk(ch, lop, fill):
            def fstep(i, f):
                d = ch[1, pl.ds(i * 16, 16)]
                sv = ch[0, pl.ds(i * 16, 16)]
                loc = d - lop
                m = (loc >= 0) & (loc < HNPW)
                plsc.store_compressed(msrc.at[pl.ds(f, 16)], sv, mask=m)
                plsc.store_compressed(mloc.at[pl.ds(f, 16)], loc, mask=m)
                return f + jnp.sum(jnp.where(m, 1, 0))

            return lax.fori_loop(0, CH // 16, fstep, fill)

        def step(ch, sem, ci, nxt_ch, nxt_sem, has_next, lop, carry):
            head, fill, pending = carry
            pltpu.make_async_copy(ei_hbm.at[:, pl.ds(0, CH)], ch, sem).wait()
            if has_next:
                pltpu.async_copy(
                    ei_hbm.at[:, pl.ds((ci + 1) * CH, CH)], nxt_ch, nxt_sem)

            @pl.when(pending > 0)
            def _():
                wait_gathers()
                process(head - G, G)

            @pl.when(head > 0)
            def _():
                def move(k, _):
                    vs = msrc[pl.ds(head + k * 16, 16)]
                    vl = mloc[pl.ds(head + k * 16, 16)]
                    msrc[pl.ds(k * 16, 16)] = vs
                    mloc[pl.ds(k * 16, 16)] = vl
                    return 0

                lax.fori_loop(0, (fill + 15) // 16, move, 0)

            fill = filter_chunk(ch, lop, fill)

            def dcond(c2):
                return c2[1] >= 2 * G

            def dbody(c2):
                q, f2 = c2
                fire(q)
                wait_gathers()
                process(q, G)
                return (q + G, f2 - G)

            qoff, fill = lax.while_loop(dcond, dbody, (0, fill))

            fire_p = (fill >= G).astype(jnp.int32)

            @pl.when(fire_p > 0)
            def _():
                fire(qoff)

            return (qoff + fire_p * G, fill - fire_p * G, fire_p)

        def halfpass(p):
            lop = lo0 + p * HNPW

            @pl.loop(0, HNPW)
            def _(i):
                for k2 in range(8):
                    sl = pl.ds(k2 * 16, 16)
                    acc_s[i, sl] = zero16
                    acc_m[i, sl] = ninf16
                    acc_c[i, sl] = zero16

            pltpu.async_copy(ei_hbm.at[:, pl.ds(0, CH)], chA, sA)

            def pair(i, carry):
                carry = step(chA, sA, 2 * i, chB, sB, True, lop, carry)
                carry = step(chB, sB, 2 * i + 1, chA, sA, None, lop, carry)
                return carry

            def pair_mid(i, carry):
                carry = step(chA, sA, 2 * i, chB, sB, True, lop, carry)
                carry = step(chB, sB, 2 * i + 1, chA, sA, True, lop, carry)
                return carry

            carry = lax.fori_loop(0, NCH // 2 - 1, pair_mid, (0, 0, 0))
            head, fill, pending = pair(NCH // 2 - 1, carry)

            @pl.when(pending > 0)
            def _():
                wait_gathers()
                process(head - G, G)

            @pl.when(fill > 0)
            def _():
                fire(head)
                wait_gathers()
                process(head, fill)

            pltpu.sync_copy(acc_s, hs_hbm.at[pl.ds(lop, HNPW), :])
            pltpu.sync_copy(acc_m, hm_hbm.at[pl.ds(lop, HNPW), :])
            pltpu.sync_copy(acc_c, cs_hbm.at[pl.ds(lop, HNPW), :])

        halfpass(0)
        halfpass(1)

    return sc_kernel(ei, h, c)


def _dense_body(hs_ref, hm_ref, cs_ref, x_ref, c_ref,
                wi_ref, ui_ref, bi_ref, ufw_ref, ufb_ref,
                hnew_ref, cnew_ref):
    hm_raw = hm_ref[...]
    has_msg = hm_raw[:, 0:1] > -jnp.inf
    hm = jnp.where(has_msg, hm_raw, 0.0)
    hcomb = jnp.concatenate([hs_ref[...], hm], axis=1)
    dn = (((1,), (1,)), ((), ()))
    f = jax.nn.sigmoid(
        jax.lax.dot_general(hcomb, ufw_ref[...], dn,
                            preferred_element_type=jnp.float32)
        + ufb_ref[...])
    iou_red = jax.lax.dot_general(hcomb, ui_ref[...], dn,
                                  preferred_element_type=jnp.float32)
    iou_leaf = jax.lax.dot_general(x_ref[...], wi_ref[...], dn,
                                   preferred_element_type=jnp.float32)
    iou = jnp.where(has_msg, iou_red, iou_leaf) + bi_ref[...]
    i = jax.nn.sigmoid(iou[:, 0:128])
    o = jax.nn.sigmoid(iou[:, 128:256])
    u = jnp.tanh(iou[:, 256:384])
    c_data = jnp.where(has_msg, f * cs_ref[...], c_ref[...])
    c_new = i * u + c_data
    hnew_ref[...] = o * jnp.tanh(c_new)
    cnew_ref[...] = c_new


def _dense_apply(hs, hm, cs, x, c, W_iou, U_iou, b_iou, U_f_w, U_f_b):
    B = 400
    grid = (N // B,)
    row_spec = pl.BlockSpec((B, H), lambda i: (i, 0))
    full = lambda a: pl.BlockSpec(a.shape, lambda i: tuple(0 for _ in a.shape))
    return pl.pallas_call(
        _dense_body,
        grid=grid,
        in_specs=[row_spec, row_spec, row_spec, row_spec, row_spec,
                  full(W_iou), full(U_iou), full(b_iou), full(U_f_w),
                  pl.BlockSpec((1, H), lambda i: (0, 0))],
        out_specs=[row_spec, row_spec],
        out_shape=[jax.ShapeDtypeStruct((N, H), jnp.float32),
                   jax.ShapeDtypeStruct((N, H), jnp.float32)],
    )(hs, hm, cs, x, c, W_iou, U_iou, b_iou, U_f_w,
      U_f_b.reshape(1, H))


def kernel(x, h, c, edge_index, W_iou, U_iou, b_iou, U_f_w, U_f_b):
    hs, hm, cs = _segment_reduce_sc(edge_index, h, c)
    return _dense_apply(hs[:N], hm[:N], cs[:N], x, c,
                        W_iou, U_iou, b_iou, U_f_w, U_f_b)


# speedup vs baseline: 2.4458x; 1.0647x over previous
import jax
import jax.numpy as jnp
from jax import lax
from jax.experimental import pallas as pl
from jax.experimental.pallas import tpu as pltpu
from jax.experimental.pallas import tpu_sc as plsc

N = 10000
E = 320000
H = 128
NW = 32
NPW = 320
HNPW = 160
NPAD = NW * NPW
CH = 1280
NCH = E // CH
G = 64
MB = 1504


def _segment_reduce_sc(ei, h, c):
    mesh = plsc.VectorSubcoreMesh(core_axis_name="c", subcore_axis_name="s")

    @pl.kernel(
        out_type=[
            jax.ShapeDtypeStruct((NPAD, H), jnp.float32),
            jax.ShapeDtypeStruct((NPAD, H), jnp.float32),
            jax.ShapeDtypeStruct((NPAD, H), jnp.float32),
        ],
        mesh=mesh,
        compiler_params=pltpu.CompilerParams(needs_layout_passes=False),
        scratch_types=[
            pltpu.VMEM((HNPW, H), jnp.float32),
            pltpu.VMEM((HNPW, H), jnp.float32),
            pltpu.VMEM((HNPW, H), jnp.float32),
            pltpu.VMEM((2, CH), jnp.int32),
            pltpu.VMEM((2, CH), jnp.int32),
            pltpu.VMEM((MB,), jnp.int32),
            pltpu.VMEM((MB,), jnp.int32),
            pltpu.VMEM((G, H), jnp.float32),
            pltpu.VMEM((G, H), jnp.float32),
            pltpu.VMEM((G, H), jnp.float32),
            pltpu.VMEM((G, H), jnp.float32),
            pltpu.VMEM((G,), jnp.int32),
            pltpu.VMEM((G,), jnp.int32),
            pltpu.VMEM((G,), jnp.int32),
            pltpu.VMEM((G,), jnp.int32),
            pltpu.SemaphoreType.DMA,
            pltpu.SemaphoreType.DMA,
            pltpu.SemaphoreType.DMA,
            pltpu.SemaphoreType.DMA,
            pltpu.SemaphoreType.DMA,
            pltpu.SemaphoreType.DMA,
        ],
    )
    def sc_kernel(ei_hbm, h_hbm, c_hbm, hs_hbm, hm_hbm, cs_hbm,
                  acc_s, acc_m, acc_c, chA, chB, msrc, mloc,
                  rowhA, rowcA, rowhB, rowcB, gidxA, glocA, gidxB, glocB,
                  sghA, sgcA, sghB, sgcB, sA, sB):
        s = lax.axis_index("s")
        wid = s * 2 + lax.axis_index("c")
        lo0 = wid * NPW
        slotA = (rowhA, rowcA, gidxA, glocA, sghA, sgcA)
        slotB = (rowhB, rowcB, gidxB, glocB, sghB, sgcB)

        zero16 = jnp.zeros((16,), jnp.float32)
        ninf16 = jnp.full((16,), -jnp.inf, jnp.float32)
        zero16i = jnp.zeros((16,), jnp.int32)
        lane = lax.iota(jnp.int32, 16)

        @pl.loop(0, MB, step=16)
        def _(i):
            msrc[pl.ds(i, 16)] = zero16i
            mloc[pl.ds(i, 16)] = zero16i

        def lane_at(v, j):
            idx = jnp.zeros((16,), jnp.int32) + j
            bc = lax.gather(
                v, idx[:, None],
                dimension_numbers=lax.GatherDimensionNumbers(
                    offset_dims=(), collapsed_slice_dims=(0,),
                    start_index_map=(0,)),
                slice_sizes=(1,),
                mode=lax.GatherScatterMode.PROMISE_IN_BOUNDS)
            return bc[0]

        def process(slot, count):
            rowh, rowc, gidx, gloc, sgh, sgc = slot

            def kgroup(k, _):
                lvec = gloc[pl.ds(k * 16, 16)]

                def ebody(j, _):
                    dloc = lane_at(lvec, j)
                    r = k * 16 + j
                    for k2 in range(8):
                        sl = pl.ds(k2 * 16, 16)
                        rh = rowh[r, sl]
                        plsc.addupdate(acc_s.at[dloc, sl], rh)
                        acc_m[dloc, sl] = jnp.maximum(acc_m[dloc, sl], rh)
                        plsc.addupdate(acc_c.at[dloc, sl], rowc[r, sl])
                    return 0

                lax.fori_loop(0, jnp.clip(count - k * 16, 0, 16), ebody, 0)
                return 0

            lax.fori_loop(0, G // 16, kgroup, 0)

        def stage_fire(slot, off):
            rowh, rowc, gidx, gloc, sgh, sgc = slot
            off = pl.multiple_of(off, 16)

            @pl.loop(0, G, step=16)
            def _(k):
                gidx[pl.ds(k, 16)] = msrc[pl.ds(off + k, 16)]
                gloc[pl.ds(k, 16)] = mloc[pl.ds(off + k, 16)]

            pltpu.async_copy(h_hbm.at[gidx], rowh, sgh)
            pltpu.async_copy(c_hbm.at[gidx], rowc, sgc)

        def wait_slot(slot):
            rowh, rowc, gidx, gloc, sgh, sgc = slot
            pltpu.make_async_copy(h_hbm.at[gidx], rowh, sgh).wait()
            pltpu.make_async_copy(c_hbm.at[gidx], rowc, sgc).wait()

        def filter_chunk(ch, lop, fill):
            def fstep(i, f):
                d = ch[1, pl.ds(i * 16, 16)]
                sv = ch[0, pl.ds(i * 16, 16)]
                loc = d - lop
                m = (loc >= 0) & (loc < HNPW)
                plsc.store_compressed(msrc.at[pl.ds(f, 16)], sv, mask=m)
                plsc.store_compressed(mloc.at[pl.ds(f, 16)], loc, mask=m)
                return f + plsc.all_reduce_population_count(m)[0]

            return lax.fori_loop(0, CH // 16, fstep, fill)

        def step(ch, sem, ci, nxt_ch, nxt_sem, has_next, slot, pend,
                 lop, head, fill):
            pltpu.make_async_copy(ei_hbm.at[:, pl.ds(0, CH)], ch, sem).wait()
            if has_next:
                pltpu.async_copy(
                    ei_hbm.at[:, pl.ds((ci + 1) * CH, CH)], nxt_ch, nxt_sem)

            @pl.when(pend > 0)
            def _():
                wait_slot(slot)
                process(slot, G)

            @pl.when(head > 0)
            def _():
                hd = pl.multiple_of(head, 16)

                def move(k, _):
                    vs = msrc[pl.ds(hd + k * 16, 16)]
                    vl = mloc[pl.ds(hd + k * 16, 16)]
                    msrc[pl.ds(k * 16, 16)] = vs
                    mloc[pl.ds(k * 16, 16)] = vl
                    return 0

                lax.fori_loop(0, (fill + 15) // 16, move, 0)

            fill = filter_chunk(ch, lop, fill)

            def dcond(c2):
                return c2[1] >= 2 * G

            def dbody(c2):
                q, f2 = c2
                stage_fire(slot, q)
                wait_slot(slot)
                process(slot, G)
                return (q + G, f2 - G)

            qoff, fill = lax.while_loop(dcond, dbody, (0, fill))

            fire_p = (fill >= G).astype(jnp.int32)

            @pl.when(fire_p > 0)
            def _():
                stage_fire(slot, qoff)

            return qoff + fire_p * G, fill - fire_p * G, fire_p

        def halfpass(p):
            lop = lo0 + p * HNPW

            @pl.loop(0, HNPW)
            def _(i):
                for k2 in range(8):
                    sl = pl.ds(k2 * 16, 16)
                    acc_s[i, sl] = zero16
                    acc_m[i, sl] = ninf16
                    acc_c[i, sl] = zero16

            pltpu.async_copy(ei_hbm.at[:, pl.ds(0, CH)], chA, sA)

            def pair(i, carry, last):
                head, fill, pA, pB = carry
                head, fill, pA = step(chA, sA, 2 * i, chB, sB, True,
                                      slotA, pA, lop, head, fill)
                head, fill, pB = step(chB, sB, 2 * i + 1, chA, sA, not last,
                                      slotB, pB, lop, head, fill)
                return (head, fill, pA, pB)

            carry = lax.fori_loop(0, NCH // 2 - 1,
                                  lambda i, cr: pair(i, cr, False),
                                  (0, 0, 0, 0))
            head, fill, pA, pB = pair(NCH // 2 - 1, carry, True)

            @pl.when(pA > 0)
            def _():
                wait_slot(slotA)
                process(slotA, G)

            @pl.when(pB > 0)
            def _():
                wait_slot(slotB)
                process(slotB, G)

            @pl.when(fill > 0)
            def _():
                stage_fire(slotA, head)
                wait_slot(slotA)
                process(slotA, fill)

            pltpu.sync_copy(acc_s, hs_hbm.at[pl.ds(lop, HNPW), :])
            pltpu.sync_copy(acc_m, hm_hbm.at[pl.ds(lop, HNPW), :])
            pltpu.sync_copy(acc_c, cs_hbm.at[pl.ds(lop, HNPW), :])

        halfpass(0)
        halfpass(1)

    return sc_kernel(ei, h, c)


def _dense_body(hs_ref, hm_ref, cs_ref, x_ref, c_ref,
                wi_ref, ui_ref, bi_ref, ufw_ref, ufb_ref,
                hnew_ref, cnew_ref):
    hm_raw = hm_ref[...]
    has_msg = hm_raw[:, 0:1] > -jnp.inf
    hm = jnp.where(has_msg, hm_raw, 0.0)
    hcomb = jnp.concatenate([hs_ref[...], hm], axis=1)
    dn = (((1,), (1,)), ((), ()))
    f = jax.nn.sigmoid(
        jax.lax.dot_general(hcomb, ufw_ref[...], dn,
                            preferred_element_type=jnp.float32)
        + ufb_ref[...])
    iou_red = jax.lax.dot_general(hcomb, ui_ref[...], dn,
                                  preferred_element_type=jnp.float32)
    iou_leaf = jax.lax.dot_general(x_ref[...], wi_ref[...], dn,
                                   preferred_element_type=jnp.float32)
    iou = jnp.where(has_msg, iou_red, iou_leaf) + bi_ref[...]
    i = jax.nn.sigmoid(iou[:, 0:128])
    o = jax.nn.sigmoid(iou[:, 128:256])
    u = jnp.tanh(iou[:, 256:384])
    c_data = jnp.where(has_msg, f * cs_ref[...], c_ref[...])
    c_new = i * u + c_data
    hnew_ref[...] = o * jnp.tanh(c_new)
    cnew_ref[...] = c_new


def _dense_apply(hs, hm, cs, x, c, W_iou, U_iou, b_iou, U_f_w, U_f_b):
    B = 400
    grid = (N // B,)
    row_spec = pl.BlockSpec((B, H), lambda i: (i, 0))
    full = lambda a: pl.BlockSpec(a.shape, lambda i: tuple(0 for _ in a.shape))
    return pl.pallas_call(
        _dense_body,
        grid=grid,
        in_specs=[row_spec, row_spec, row_spec, row_spec, row_spec,
                  full(W_iou), full(U_iou), full(b_iou), full(U_f_w),
                  pl.BlockSpec((1, H), lambda i: (0, 0))],
        out_specs=[row_spec, row_spec],
        out_shape=[jax.ShapeDtypeStruct((N, H), jnp.float32),
                   jax.ShapeDtypeStruct((N, H), jnp.float32)],
    )(hs, hm, cs, x, c, W_iou, U_iou, b_iou, U_f_w,
      U_f_b.reshape(1, H))


def kernel(x, h, c, edge_index, W_iou, U_iou, b_iou, U_f_w, U_f_b):
    hs, hm, cs = _segment_reduce_sc(edge_index, h, c)
    return _dense_apply(hs[:N], hm[:N], cs[:N], x, c,
                        W_iou, U_iou, b_iou, U_f_w, U_f_b)


# speedup vs baseline: 2.8614x; 1.1699x over previous
import jax
import jax.numpy as jnp
from jax import lax
from jax.experimental import pallas as pl
from jax.experimental.pallas import tpu as pltpu
from jax.experimental.pallas import tpu_sc as plsc

N = 10000
E = 320000
H = 128
NW = 32
NPW = 320
HNPW = 160
NPAD = NW * NPW
CH = 1280
NCH = E // CH
G = 64
MB = 1504
SH = 16 * HNPW
DUMP = SH


def _segment_reduce_sc(ei, h, c):
    mesh = plsc.VectorSubcoreMesh(core_axis_name="c", subcore_axis_name="s")

    @pl.kernel(
        out_type=[
            jax.ShapeDtypeStruct((NPAD, H), jnp.float32),
            jax.ShapeDtypeStruct((NPAD, H), jnp.float32),
            jax.ShapeDtypeStruct((NPAD, H), jnp.float32),
        ],
        mesh=mesh,
        compiler_params=pltpu.CompilerParams(needs_layout_passes=False),
        scratch_types=[
            pltpu.VMEM((HNPW, H), jnp.float32),
            pltpu.VMEM((HNPW, H), jnp.float32),
            pltpu.VMEM((2, CH), jnp.int32),
            pltpu.VMEM((2, CH), jnp.int32),
            pltpu.VMEM((MB,), jnp.int32),
            pltpu.VMEM((MB,), jnp.int32),
            pltpu.VMEM((G, H), jnp.float32),
            pltpu.VMEM((G, H), jnp.float32),
            pltpu.VMEM((G, H), jnp.float32),
            pltpu.VMEM((G, H), jnp.float32),
            pltpu.VMEM((G,), jnp.int32),
            pltpu.VMEM((G,), jnp.int32),
            pltpu.VMEM((G,), jnp.int32),
            pltpu.VMEM((G,), jnp.int32),
            pltpu.VMEM((G,), jnp.int32),
            pltpu.VMEM((G,), jnp.int32),
            pltpu.VMEM((G, H), jnp.float32),
            pltpu.VMEM_SHARED((SH + 1, H), jnp.float32),
            pltpu.SemaphoreType.DMA,
            pltpu.SemaphoreType.DMA,
            pltpu.SemaphoreType.DMA,
            pltpu.SemaphoreType.DMA,
            pltpu.SemaphoreType.DMA,
            pltpu.SemaphoreType.DMA,
            pltpu.SemaphoreType.DMA,
            pltpu.SemaphoreType.DMA,
        ],
    )
    def sc_kernel(ei_hbm, h_hbm, c_hbm, hs_hbm, hm_hbm, cs_hbm,
                  acc_s, acc_m, chA, chB, msrc, mloc,
                  rowhA, rowcA, rowhB, rowcB,
                  gidxA, glocA, gsidA, gidxB, glocB, gsidB,
                  zrow, csum_sh,
                  sghA, sgcA, sscA, sghB, sgcB, sscB, sA, sB):
        s = lax.axis_index("s")
        wid = s * 2 + lax.axis_index("c")
        lo0 = wid * NPW
        sbase = s * HNPW
        slotA = (rowhA, rowcA, gidxA, glocA, gsidA, sghA, sgcA, sscA)
        slotB = (rowhB, rowcB, gidxB, glocB, gsidB, sghB, sgcB, sscB)

        zero16 = jnp.zeros((16,), jnp.float32)
        ninf16 = jnp.full((16,), -jnp.inf, jnp.float32)
        zero16i = jnp.zeros((16,), jnp.int32)
        lane = lax.iota(jnp.int32, 16)

        @pl.loop(0, MB, step=16)
        def _(i):
            msrc[pl.ds(i, 16)] = zero16i
            mloc[pl.ds(i, 16)] = zero16i

        @pl.loop(0, G)
        def _(i):
            for k2 in range(8):
                zrow[i, pl.ds(k2 * 16, 16)] = zero16

        def lane_bc(v, j):
            idx = jnp.zeros((16,), jnp.int32) + j
            return lax.gather(
                v, idx[:, None],
                dimension_numbers=lax.GatherDimensionNumbers(
                    offset_dims=(), collapsed_slice_dims=(0,),
                    start_index_map=(0,)),
                slice_sizes=(1,),
                mode=lax.GatherScatterMode.PROMISE_IN_BOUNDS)

        cols = [lane + k2 * 16 for k2 in range(8)]

        def process(slot, count):
            rowh, rowc, gidx, gloc, gsid, sgh, sgc, ssc = slot

            def kgroup(k, _):
                lvec = gloc[pl.ds(k * 16, 16)]

                def ebody(j, _):
                    rowv = lane_bc(lvec, j)
                    r = k * 16 + j
                    for k2 in range(8):
                        rh = rowh[r, pl.ds(k2 * 16, 16)]
                        plsc.addupdate_scatter(acc_s, [rowv, cols[k2]], rh)
                        mx = plsc.load_gather(acc_m, [rowv, cols[k2]])
                        plsc.store_scatter(acc_m, [rowv, cols[k2]],
                                           jnp.maximum(mx, rh))
                    return 0

                lax.fori_loop(0, jnp.clip(count - k * 16, 0, 16), ebody, 0)
                return 0

            lax.fori_loop(0, G // 16, kgroup, 0)

        def stage_fire(slot, off, count):
            rowh, rowc, gidx, gloc, gsid, sgh, sgc, ssc = slot
            off = pl.multiple_of(off, 16)

            @pl.loop(0, G, step=16)
            def _(k):
                lv = mloc[pl.ds(off + k, 16)]
                gidx[pl.ds(k, 16)] = msrc[pl.ds(off + k, 16)]
                gloc[pl.ds(k, 16)] = lv
                gsid[pl.ds(k, 16)] = jnp.where(lane + k < count,
                                               lv + sbase, DUMP)

            pltpu.async_copy(h_hbm.at[gidx], rowh, sgh)
            pltpu.async_copy(c_hbm.at[gidx], rowc, sgc)

        def consume(slot, count):
            rowh, rowc, gidx, gloc, gsid, sgh, sgc, ssc = slot
            pltpu.make_async_copy(h_hbm.at[gidx], rowh, sgh).wait()
            pltpu.make_async_copy(c_hbm.at[gidx], rowc, sgc).wait()
            cp = pltpu.async_copy(rowc, csum_sh.at[gsid], ssc, add=True)
            process(slot, count)
            cp.wait()

        def filter_chunk(ch, lop, fill):
            def fstep(i, f):
                d = ch[1, pl.ds(i * 16, 16)]
                sv = ch[0, pl.ds(i * 16, 16)]
                loc = d - lop
                m = (loc >= 0) & (loc < HNPW)
                plsc.store_compressed(msrc.at[pl.ds(f, 16)], sv, mask=m)
                plsc.store_compressed(mloc.at[pl.ds(f, 16)], loc, mask=m)
                return f + plsc.all_reduce_population_count(m)[0]

            return lax.fori_loop(0, CH // 16, fstep, fill)

        def step(ch, sem, ci, nxt_ch, nxt_sem, has_next, slot, pend,
                 lop, head, fill):
            pltpu.make_async_copy(ei_hbm.at[:, pl.ds(0, CH)], ch, sem).wait()
            if has_next:
                pltpu.async_copy(
                    ei_hbm.at[:, pl.ds((ci + 1) * CH, CH)], nxt_ch, nxt_sem)

            @pl.when(pend > 0)
            def _():
                consume(slot, G)

            @pl.when(head > 0)
            def _():
                hd = pl.multiple_of(head, 16)

                def move(k, _):
                    vs = msrc[pl.ds(hd + k * 16, 16)]
                    vl = mloc[pl.ds(hd + k * 16, 16)]
                    msrc[pl.ds(k * 16, 16)] = vs
                    mloc[pl.ds(k * 16, 16)] = vl
                    return 0

                lax.fori_loop(0, (fill + 15) // 16, move, 0)

            fill = filter_chunk(ch, lop, fill)

            def dcond(c2):
                return c2[1] >= 2 * G

            def dbody(c2):
                q, f2 = c2
                stage_fire(slot, q, G)
                consume(slot, G)
                return (q + G, f2 - G)

            qoff, fill = lax.while_loop(dcond, dbody, (0, fill))

            fire_p = (fill >= G).astype(jnp.int32)

            @pl.when(fire_p > 0)
            def _():
                stage_fire(slot, qoff, G)

            return qoff + fire_p * G, fill - fire_p * G, fire_p

        def halfpass(p):
            lop = lo0 + p * HNPW

            @pl.loop(0, HNPW)
            def _(i):
                for k2 in range(8):
                    sl = pl.ds(k2 * 16, 16)
                    acc_s[i, sl] = zero16
                    acc_m[i, sl] = ninf16

            pltpu.sync_copy(zrow, csum_sh.at[pl.ds(sbase, G), :])
            pltpu.sync_copy(zrow, csum_sh.at[pl.ds(sbase + G, G), :])
            pltpu.sync_copy(zrow.at[pl.ds(0, 32), :],
                            csum_sh.at[pl.ds(sbase + 2 * G, 32), :])

            pltpu.async_copy(ei_hbm.at[:, pl.ds(0, CH)], chA, sA)

            def pair(i, carry, last):
                head, fill, pA, pB = carry
                head, fill, pA = step(chA, sA, 2 * i, chB, sB, True,
                                      slotA, pA, lop, head, fill)
                head, fill, pB = step(chB, sB, 2 * i + 1, chA, sA, not last,
                                      slotB, pB, lop, head, fill)
                return (head, fill, pA, pB)

            carry = lax.fori_loop(0, NCH // 2 - 1,
                                  lambda i, cr: pair(i, cr, False),
                                  (0, 0, 0, 0))
            head, fill, pA, pB = pair(NCH // 2 - 1, carry, True)

            @pl.when(pA > 0)
            def _():
                consume(slotA, G)

            @pl.when(pB > 0)
            def _():
                consume(slotB, G)

            @pl.when(fill > 0)
            def _():
                stage_fire(slotA, head, fill)
                consume(slotA, fill)

            pltpu.sync_copy(acc_s, hs_hbm.at[pl.ds(lop, HNPW), :])
            pltpu.sync_copy(acc_m, hm_hbm.at[pl.ds(lop, HNPW), :])
            pltpu.sync_copy(csum_sh.at[pl.ds(sbase, HNPW), :],
                            cs_hbm.at[pl.ds(lop, HNPW), :])

        halfpass(0)
        halfpass(1)

    return sc_kernel(ei, h, c)


def _dense_body(hs_ref, hm_ref, cs_ref, x_ref, c_ref,
                wi_ref, ui_ref, bi_ref, ufw_ref, ufb_ref,
                hnew_ref, cnew_ref):
    hm_raw = hm_ref[...]
    has_msg = hm_raw[:, 0:1] > -jnp.inf
    hm = jnp.where(has_msg, hm_raw, 0.0)
    hcomb = jnp.concatenate([hs_ref[...], hm], axis=1)
    dn = (((1,), (1,)), ((), ()))
    f = jax.nn.sigmoid(
        jax.lax.dot_general(hcomb, ufw_ref[...], dn,
                            preferred_element_type=jnp.float32)
        + ufb_ref[...])
    iou_red = jax.lax.dot_general(hcomb, ui_ref[...], dn,
                                  preferred_element_type=jnp.float32)
    iou_leaf = jax.lax.dot_general(x_ref[...], wi_ref[...], dn,
                                   preferred_element_type=jnp.float32)
    iou = jnp.where(has_msg, iou_red, iou_leaf) + bi_ref[...]
    i = jax.nn.sigmoid(iou[:, 0:128])
    o = jax.nn.sigmoid(iou[:, 128:256])
    u = jnp.tanh(iou[:, 256:384])
    c_data = jnp.where(has_msg, f * cs_ref[...], c_ref[...])
    c_new = i * u + c_data
    hnew_ref[...] = o * jnp.tanh(c_new)
    cnew_ref[...] = c_new


def _dense_apply(hs, hm, cs, x, c, W_iou, U_iou, b_iou, U_f_w, U_f_b):
    B = 400
    grid = (N // B,)
    row_spec = pl.BlockSpec((B, H), lambda i: (i, 0))
    full = lambda a: pl.BlockSpec(a.shape, lambda i: tuple(0 for _ in a.shape))
    return pl.pallas_call(
        _dense_body,
        grid=grid,
        in_specs=[row_spec, row_spec, row_spec, row_spec, row_spec,
                  full(W_iou), full(U_iou), full(b_iou), full(U_f_w),
                  pl.BlockSpec((1, H), lambda i: (0, 0))],
        out_specs=[row_spec, row_spec],
        out_shape=[jax.ShapeDtypeStruct((N, H), jnp.float32),
                   jax.ShapeDtypeStruct((N, H), jnp.float32)],
    )(hs, hm, cs, x, c, W_iou, U_iou, b_iou, U_f_w,
      U_f_b.reshape(1, H))


def kernel(x, h, c, edge_index, W_iou, U_iou, b_iou, U_f_w, U_f_b):
    hs, hm, cs = _segment_reduce_sc(edge_index, h, c)
    return _dense_apply(hs[:N], hm[:N], cs[:N], x, c,
                        W_iou, U_iou, b_iou, U_f_w, U_f_b)
